# baseline (speedup 1.0000x reference)
import functools
import math

import jax
import jax.numpy as jnp
from jax import lax
from jax.experimental import pallas as pl
from jax.experimental.pallas import tpu as pltpu
from jax.experimental.pallas import tpu_sc as plsc

D_MODEL = 64
SCALE = math.sqrt(D_MODEL)
LANES = 16
NUM_CORES = 2
NUM_SUBCORES = 16
NUM_WORKERS = NUM_CORES * NUM_SUBCORES
CHUNK = 128
NBUF = 4


def _emb_body(idx_hbm, tab_hbm, out_hbm, idx_v, bufs, gsems, ssems,
              *, b_per_w):
  nsteps = b_per_w // CHUNK
  wid = lax.axis_index("s") * NUM_CORES + lax.axis_index("c")
  base = wid * b_per_w

  pltpu.sync_copy(idx_hbm.at[pl.ds(base, b_per_w)], idx_v)

  def start_gather(g, b):
    pltpu.async_copy(tab_hbm.at[idx_v.at[pl.ds(g * CHUNK, CHUNK)]],
                     bufs[b], gsems[b])

  def wait_gather(b):
    pltpu.make_async_copy(tab_hbm.at[idx_v.at[pl.ds(0, CHUNK)]],
                          bufs[b], gsems[b]).wait()

  def start_scatter(g, b):
    pltpu.async_copy(bufs[b], out_hbm.at[pl.ds(base + g * CHUNK, CHUNK)],
                     ssems[b])

  def wait_scatter(b):
    pltpu.make_async_copy(bufs[b], out_hbm.at[pl.ds(0, CHUNK)],
                          ssems[b]).wait()

  def scale(b):
    buf = bufs[b]

    def row_body(i, _):
      for j in range(D_MODEL // LANES):
        sl = (i, pl.ds(j * LANES, LANES))
        buf[sl] = buf[sl] * SCALE
      return 0

    lax.fori_loop(0, CHUNK, row_body, 0)

  for b in range(NBUF):
    start_gather(b, b)

  def step(gg, _):
    for b in range(NBUF):
      g = gg * NBUF + b
      wait_gather(b)
      scale(b)
      start_scatter(g, b)
      wait_scatter(b)
      start_gather(g + NBUF, b)
    return 0

  lax.fori_loop(0, nsteps // NBUF - 1, step, 0)

  for b in range(NBUF):
    g = nsteps - NBUF + b
    wait_gather(b)
    scale(b)
    start_scatter(g, b)
    wait_scatter(b)


@jax.jit
def _emb_lookup(idx_flat, lut):
  n = idx_flat.shape[0]
  assert n % (NUM_WORKERS * CHUNK) == 0
  b_per_w = n // NUM_WORKERS
  mesh = plsc.VectorSubcoreMesh(
      core_axis_name="c", subcore_axis_name="s",
      num_cores=NUM_CORES, num_subcores=NUM_SUBCORES)
  body = functools.partial(_emb_body, b_per_w=b_per_w)
  return pl.kernel(
      body,
      out_type=jax.ShapeDtypeStruct((n, D_MODEL), jnp.float32),
      mesh=mesh,
      scratch_types=[
          pltpu.VMEM((b_per_w,), jnp.int32),
          [pltpu.VMEM((CHUNK, D_MODEL), jnp.float32) for _ in range(NBUF)],
          [pltpu.SemaphoreType.DMA for _ in range(NBUF)],
          [pltpu.SemaphoreType.DMA for _ in range(NBUF)],
      ],
      compiler_params=pltpu.CompilerParams(use_tc_tiling_on_sc=False),
      name="sc_embedding_lookup",
  )(idx_flat, lut)


def kernel(x, lut):
  idx_flat = x.reshape(-1).astype(jnp.int32)
  out = _emb_lookup(idx_flat, lut)
  return out.reshape(x.shape + (D_MODEL,))
